# baseline (speedup 1.0000x reference)
import functools

import jax
import jax.numpy as jnp
from jax import lax
from jax.experimental import pallas as pl
from jax.experimental.pallas import tpu as pltpu
from jax.experimental.pallas import tpu_sc as plsc

B = 4096
MARGIN = 0.5
L = 16
NC = 2
NS = 16
NW = NC * NS
SPLIT_ROWS = 1536
ROWS_PER_W = SPLIT_ROWS // NW
NCHUNK = B // L
TC_BLK = 512
INF = float("inf")

_mesh = plsc.VectorSubcoreMesh(core_axis_name="c", subcore_axis_name="s")


@functools.partial(
    pl.kernel,
    mesh=_mesh,
    out_type=[
        jax.ShapeDtypeStruct((NW, L), jnp.float32),
        jax.ShapeDtypeStruct((NW, L), jnp.float32),
    ],
    scratch_types=[
        pltpu.VMEM((B,), jnp.float32),
        pltpu.VMEM((B,), jnp.float32),
        pltpu.VMEM((B + L,), jnp.float32),
        pltpu.VMEM((B,), jnp.float32),
        pltpu.VMEM((B,), jnp.float32),
        pltpu.VMEM((L,), jnp.float32),
        pltpu.VMEM((L,), jnp.float32),
    ],
)
def _pairwise_sc(p_hbm, t_hbm, e_hbm, num_hbm, den_hbm,
                 pv, tv, ev, t0v, t1v, nv, dv):
    cid = lax.axis_index("c")
    sid = lax.axis_index("s")
    wid = sid * NC + cid

    pltpu.sync_copy(p_hbm, pv)
    pltpu.sync_copy(t_hbm, tv)
    pltpu.sync_copy(e_hbm, ev.at[pl.ds(0, B)])

    lanes = lax.iota(jnp.int32, L)
    zeros = jnp.zeros((L,), jnp.float32)
    ones = jnp.ones((L,), jnp.float32)
    infs = jnp.full((L,), INF, jnp.float32)

    def prep_body(c, dummy):
        b2 = c * L
        t_c = tv[pl.ds(b2, L)]
        e_c = ev[pl.ds(b2, L)] > 0.0
        t0v[pl.ds(b2, L)] = jnp.where(e_c, t_c, infs)
        t1v[pl.ds(b2, L)] = jnp.where(e_c, infs, t_c)
        return dummy

    lax.fori_loop(0, NCHUNK, prep_body, 0)

    def contrib(p_i, t_i, e_i, p_j, t_j, e_j):
        lt = t_i < t_j
        gt = t_j < t_i
        d = p_i - p_j
        yd = jnp.where(lt, d, -d)
        h = jnp.maximum(MARGIN - yd, 0.0)
        m = jnp.where(lt, e_i, jnp.where(gt, e_j, e_i * e_j))
        return h * m, m

    nv[...] = zeros
    dv[...] = zeros

    def row_body(k, dummy):
        i = wid + NW * k
        cc = i // L
        off = i - cc * L
        base = cc * L
        p_c = pv[pl.ds(base, L)]
        t_c = tv[pl.ds(base, L)]
        e_c = ev[pl.ds(base, L)]
        lane = jnp.full((L,), off, dtype=jnp.int32)
        p_i = p_c.at[lane].get(mode="promise_in_bounds")
        t_i = t_c.at[lane].get(mode="promise_in_bounds")
        e_i = e_c.at[lane].get(mode="promise_in_bounds")
        mi = MARGIN - p_i
        ma = MARGIN + p_i
        hm, m = contrib(p_i, t_i, e_i, p_c, t_c, e_c)
        tri = lanes > off

        cx = ((cc + NCHUNK // 2) & (NCHUNK - 1)) * L
        hm_x, m_x = contrib(p_i, t_i, e_i,
                            pv[pl.ds(cx, L)], tv[pl.ds(cx, L)], ev[pl.ds(cx, L)])
        lim = off - jnp.where(i >= B // 2, 1, 0)
        keep_x = lanes <= lim

        ce = ((cc + NCHUNK // 2 - 1) & (NCHUNK - 1)) * L
        hm_e, m_e = contrib(p_i, t_i, e_i,
                            pv[pl.ds(ce, L)], tv[pl.ds(ce, L)], ev[pl.ds(ce, L)])

        nv[...] = (nv[...] + jnp.where(tri, hm, zeros)
                   + jnp.where(keep_x, hm_x, zeros) + hm_e)
        dv[...] = (dv[...] + jnp.where(tri, m, zeros)
                   + jnp.where(keep_x, m_x, zeros) + m_e)

        HALF = NCHUNK // 4 - 1

        def rows_with_event(_):
            @plsc.parallel_loop(1, HALF + 1, unroll=8,
                                carry=(zeros, zeros, zeros, zeros))
            def loop1(c, carry2):
                na, da, nb, db = carry2
                bA = ((cc + c) & (NCHUNK - 1)) * L
                bB = ((cc + c + HALF) & (NCHUNK - 1)) * L
                pA = pv[pl.ds(bA, L)]
                tA = tv[pl.ds(bA, L)]
                t1A = t1v[pl.ds(bA, L)]
                pB = pv[pl.ds(bB, L)]
                tB = tv[pl.ds(bB, L)]
                t1B = t1v[pl.ds(bB, L)]
                ltA = t_i < tA
                mA = t_i < t1A
                hA = jnp.maximum(jnp.where(ltA, mi + pA, ma - pA), 0.0)
                ltB = t_i < tB
                mB = t_i < t1B
                hB = jnp.maximum(jnp.where(ltB, mi + pB, ma - pB), 0.0)
                return (na + jnp.where(mA, hA, zeros),
                        da + jnp.where(mA, ones, zeros),
                        nb + jnp.where(mB, hB, zeros),
                        db + jnp.where(mB, ones, zeros))

            na, da, nb, db = loop1
            nv[...] = nv[...] + na + nb
            dv[...] = dv[...] + da + db

        def rows_without_event(_):
            @plsc.parallel_loop(1, HALF + 1, unroll=8,
                                carry=(zeros, zeros, zeros, zeros))
            def loop0(c, carry2):
                na, da, nb, db = carry2
                bA = ((cc + c) & (NCHUNK - 1)) * L
                bB = ((cc + c + HALF) & (NCHUNK - 1)) * L
                pA = pv[pl.ds(bA, L)]
                t0A = t0v[pl.ds(bA, L)]
                pB = pv[pl.ds(bB, L)]
                t0B = t0v[pl.ds(bB, L)]
                mA = t0A < t_i
                hA = jnp.maximum(ma - pA, 0.0)
                mB = t0B < t_i
                hB = jnp.maximum(ma - pB, 0.0)
                return (na + jnp.where(mA, hA, zeros),
                        da + jnp.where(mA, ones, zeros),
                        nb + jnp.where(mB, hB, zeros),
                        db + jnp.where(mB, ones, zeros))

            na, da, nb, db = loop0
            nv[...] = nv[...] + na + nb
            dv[...] = dv[...] + da + db

        has_event = ev[pl.ds(i, L)][0] > 0.0
        lax.cond(has_event, rows_with_event, rows_without_event, 0)
        return dummy

    lax.fori_loop(0, ROWS_PER_W, row_body, 0)
    pltpu.sync_copy(nv, num_hbm.at[wid])
    pltpu.sync_copy(dv, den_hbm.at[wid])


def _tc_band(pr, tr, er, pcol, tcol, ecol, num_ref, den_ref):
    k = pl.program_id(0)
    p_i = pcol[...]
    t_i = tcol[...]
    e_i = ecol[...]
    p_j = pr[...]
    t_j = tr[...]
    e_j = er[...]
    i_idx = (SPLIT_ROWS + k * TC_BLK
             + lax.broadcasted_iota(jnp.int32, (TC_BLK, 1), 0))
    j_idx = lax.broadcasted_iota(jnp.int32, (1, B), 1)
    d = (j_idx - i_idx) & (B - 1)
    limit = jnp.where(i_idx >= B // 2, B // 2 - 1, B // 2)
    keep = (d > 0) & (d <= limit)
    lt = t_i < t_j
    gt = t_j < t_i
    df = p_i - p_j
    yd = jnp.where(lt, df, -df)
    h = jnp.maximum(MARGIN - yd, 0.0)
    m = jnp.where(lt, e_i * jnp.ones_like(h),
                  jnp.where(gt, e_j * jnp.ones_like(h), e_i * e_j))
    m = jnp.where(keep, m, 0.0)

    @pl.when(k == 0)
    def _():
        num_ref[0, 0] = 0.0
        den_ref[0, 0] = 0.0

    num_ref[0, 0] += jnp.sum(h * m)
    den_ref[0, 0] += jnp.sum(m)


def _final_reduce(num_ref, den_ref, ntc_ref, dtc_ref, out_ref):
    s = ((jnp.sum(num_ref[...]) + ntc_ref[0, 0])
         / (jnp.sum(den_ref[...]) + dtc_ref[0, 0]))
    out_ref[...] = jnp.full((1, 1), s, jnp.float32)


def kernel(y_hat, efs_time, efs):
    y_hat = jnp.squeeze(y_hat).astype(jnp.float32)
    efs_time = efs_time.astype(jnp.float32)
    efs_f = efs.astype(jnp.float32)
    num, den = _pairwise_sc(y_hat, efs_time, efs_f)
    pr = y_hat.reshape(1, B)
    tr = efs_time.reshape(1, B)
    er = efs_f.reshape(1, B)
    pcol = y_hat.reshape(B, 1)
    tcol = efs_time.reshape(B, 1)
    ecol = efs_f.reshape(B, 1)
    row_spec = pl.BlockSpec((1, B), lambda k: (0, 0))
    col_spec = pl.BlockSpec((TC_BLK, 1), lambda k: (SPLIT_ROWS // TC_BLK + k, 0))
    ntc, dtc = pl.pallas_call(
        _tc_band,
        grid=((B - SPLIT_ROWS) // TC_BLK,),
        in_specs=[row_spec, row_spec, row_spec, col_spec, col_spec, col_spec],
        out_specs=[
            pl.BlockSpec(memory_space=pltpu.SMEM),
            pl.BlockSpec(memory_space=pltpu.SMEM),
        ],
        out_shape=[
            jax.ShapeDtypeStruct((1, 1), jnp.float32),
            jax.ShapeDtypeStruct((1, 1), jnp.float32),
        ],
    )(pr, tr, er, pcol, tcol, ecol)
    out = pl.pallas_call(
        _final_reduce,
        in_specs=[
            pl.BlockSpec(memory_space=pltpu.VMEM),
            pl.BlockSpec(memory_space=pltpu.VMEM),
            pl.BlockSpec(memory_space=pltpu.SMEM),
            pl.BlockSpec(memory_space=pltpu.SMEM),
        ],
        out_shape=jax.ShapeDtypeStruct((1, 1), jnp.float32),
    )(num, den, ntc, dtc)
    return out[0, 0]


# speedup vs baseline: 1.1775x; 1.1775x over previous
import functools

import jax
import jax.numpy as jnp
from jax import lax
from jax.experimental import pallas as pl
from jax.experimental.pallas import tpu as pltpu
from jax.experimental.pallas import tpu_sc as plsc

B = 4096
MARGIN = 0.5
L = 16
NC = 2
NS = 16
NW = NC * NS
SPLIT_ROWS = 1536
ROWS_PER_W = SPLIT_ROWS // NW
NCHUNK = B // L
TC_BLK = 512
INF = float("inf")

_mesh = plsc.VectorSubcoreMesh(core_axis_name="c", subcore_axis_name="s")


@functools.partial(
    pl.kernel,
    mesh=_mesh,
    out_type=[
        jax.ShapeDtypeStruct((NW, L), jnp.float32),
        jax.ShapeDtypeStruct((NW, L), jnp.float32),
    ],
    scratch_types=[
        pltpu.VMEM((B,), jnp.float32),
        pltpu.VMEM((B,), jnp.float32),
        pltpu.VMEM((B + L,), jnp.float32),
        pltpu.VMEM((B,), jnp.float32),
        pltpu.VMEM((B,), jnp.float32),
        pltpu.VMEM((L,), jnp.float32),
        pltpu.VMEM((L,), jnp.float32),
    ],
)
def _pairwise_sc(p_hbm, t_hbm, e_hbm, num_hbm, den_hbm,
                 pv, tv, ev, t0v, t1v, nv, dv):
    cid = lax.axis_index("c")
    sid = lax.axis_index("s")
    wid = sid * NC + cid

    pltpu.sync_copy(p_hbm, pv)
    pltpu.sync_copy(t_hbm, tv)
    pltpu.sync_copy(e_hbm, ev.at[pl.ds(0, B)])

    lanes = lax.iota(jnp.int32, L)
    zeros = jnp.zeros((L,), jnp.float32)
    ones = jnp.ones((L,), jnp.float32)
    infs = jnp.full((L,), INF, jnp.float32)

    def prep_body(c, dummy):
        b2 = c * L
        t_c = tv[pl.ds(b2, L)]
        e_c = ev[pl.ds(b2, L)] > 0.0
        t0v[pl.ds(b2, L)] = jnp.where(e_c, t_c, infs)
        t1v[pl.ds(b2, L)] = jnp.where(e_c, infs, t_c)
        return dummy

    lax.fori_loop(0, NCHUNK, prep_body, 0)

    def contrib(p_i, t_i, e_i, p_j, t_j, e_j):
        lt = t_i < t_j
        gt = t_j < t_i
        d = p_i - p_j
        yd = jnp.where(lt, d, -d)
        h = jnp.maximum(MARGIN - yd, 0.0)
        m = jnp.where(lt, e_i, jnp.where(gt, e_j, e_i * e_j))
        return h * m, m

    nv[...] = zeros
    dv[...] = zeros

    def row_body(k, dummy):
        i = wid + NW * k
        cc = i // L
        off = i - cc * L
        base = cc * L
        p_c = pv[pl.ds(base, L)]
        t_c = tv[pl.ds(base, L)]
        e_c = ev[pl.ds(base, L)]
        lane = jnp.full((L,), off, dtype=jnp.int32)
        p_i = p_c.at[lane].get(mode="promise_in_bounds")
        t_i = t_c.at[lane].get(mode="promise_in_bounds")
        e_i = e_c.at[lane].get(mode="promise_in_bounds")
        mi = MARGIN - p_i
        ma = MARGIN + p_i
        hm, m = contrib(p_i, t_i, e_i, p_c, t_c, e_c)
        tri = lanes > off

        cx = ((cc + NCHUNK // 2) & (NCHUNK - 1)) * L
        hm_x, m_x = contrib(p_i, t_i, e_i,
                            pv[pl.ds(cx, L)], tv[pl.ds(cx, L)], ev[pl.ds(cx, L)])
        lim = off - jnp.where(i >= B // 2, 1, 0)
        keep_x = lanes <= lim

        ce = ((cc + NCHUNK // 2 - 1) & (NCHUNK - 1)) * L
        hm_e, m_e = contrib(p_i, t_i, e_i,
                            pv[pl.ds(ce, L)], tv[pl.ds(ce, L)], ev[pl.ds(ce, L)])

        nv[...] = (nv[...] + jnp.where(tri, hm, zeros)
                   + jnp.where(keep_x, hm_x, zeros) + hm_e)
        dv[...] = (dv[...] + jnp.where(tri, m, zeros)
                   + jnp.where(keep_x, m_x, zeros) + m_e)

        HALF = NCHUNK // 4 - 1

        def rows_with_event(_):
            @plsc.parallel_loop(1, HALF + 1, unroll=8,
                                carry=(zeros, zeros, zeros, zeros))
            def loop1(c, carry2):
                na, da, nb, db = carry2
                bA = ((cc + c) & (NCHUNK - 1)) * L
                bB = ((cc + c + HALF) & (NCHUNK - 1)) * L
                pA = pv[pl.ds(bA, L)]
                tA = tv[pl.ds(bA, L)]
                t1A = t1v[pl.ds(bA, L)]
                pB = pv[pl.ds(bB, L)]
                tB = tv[pl.ds(bB, L)]
                t1B = t1v[pl.ds(bB, L)]
                ltA = t_i < tA
                mA = t_i < t1A
                hA = jnp.maximum(jnp.where(ltA, mi + pA, ma - pA), 0.0)
                ltB = t_i < tB
                mB = t_i < t1B
                hB = jnp.maximum(jnp.where(ltB, mi + pB, ma - pB), 0.0)
                return (na + jnp.where(mA, hA, zeros),
                        da + jnp.where(mA, ones, zeros),
                        nb + jnp.where(mB, hB, zeros),
                        db + jnp.where(mB, ones, zeros))

            na, da, nb, db = loop1
            nv[...] = nv[...] + na + nb
            dv[...] = dv[...] + da + db

        def rows_without_event(_):
            @plsc.parallel_loop(1, HALF + 1, unroll=8,
                                carry=(zeros, zeros, zeros, zeros))
            def loop0(c, carry2):
                na, da, nb, db = carry2
                bA = ((cc + c) & (NCHUNK - 1)) * L
                bB = ((cc + c + HALF) & (NCHUNK - 1)) * L
                pA = pv[pl.ds(bA, L)]
                t0A = t0v[pl.ds(bA, L)]
                pB = pv[pl.ds(bB, L)]
                t0B = t0v[pl.ds(bB, L)]
                mA = t0A < t_i
                hA = jnp.maximum(ma - pA, 0.0)
                mB = t0B < t_i
                hB = jnp.maximum(ma - pB, 0.0)
                return (na + jnp.where(mA, hA, zeros),
                        da + jnp.where(mA, ones, zeros),
                        nb + jnp.where(mB, hB, zeros),
                        db + jnp.where(mB, ones, zeros))

            na, da, nb, db = loop0
            nv[...] = nv[...] + na + nb
            dv[...] = dv[...] + da + db

        has_event = ev[pl.ds(i, L)][0] > 0.0
        lax.cond(has_event, rows_with_event, rows_without_event, 0)
        return dummy

    lax.fori_loop(0, ROWS_PER_W, row_body, 0)
    pltpu.sync_copy(nv, num_hbm.at[wid])
    pltpu.sync_copy(dv, den_hbm.at[wid])


NPIECE = B // 2 // TC_BLK + 1


def _tc_band(*refs):
    ps = refs[0:NPIECE]
    ts = refs[NPIECE:2 * NPIECE]
    es = refs[2 * NPIECE:3 * NPIECE]
    pcol, tcol, ecol, num_ref, den_ref = refs[3 * NPIECE:]
    k = pl.program_id(0)
    base = SPLIT_ROWS + k * TC_BLK
    p_i = pcol[...]
    t_i = tcol[...]
    e_i = ecol[...]
    mi = MARGIN - p_i
    ma = MARGIN + p_i
    ei_pos = e_i > 0.0
    r = lax.broadcasted_iota(jnp.int32, (TC_BLK, 1), 0)
    lim = r + B // 2 - jnp.where(base + r >= B // 2, 1, 0)
    num_s = jnp.zeros((), jnp.float32)
    den_s = jnp.zeros((), jnp.float32)
    for c in range(NPIECE):
        pj = ps[c][...]
        tj = ts[c][...]
        ej = es[c][...]
        ebj = ej > 0.0
        t0s = jnp.where(ebj, tj, INF)
        t1s = jnp.where(ebj, INF, tj)
        s = c * TC_BLK + lax.broadcasted_iota(jnp.int32, (1, TC_BLK), 1)
        keep = (s > r) & (s <= lim)
        lt = t_i < tj
        m = ((ei_pos & (t_i < t1s)) | (~ei_pos & (t0s < t_i))) & keep
        h = jnp.maximum(jnp.where(lt, mi + pj, ma - pj), 0.0)
        num_s += jnp.sum(jnp.where(m, h, 0.0))
        den_s += jnp.sum(jnp.where(m, 1.0, 0.0))

    @pl.when(k == 0)
    def _():
        num_ref[0, 0] = 0.0
        den_ref[0, 0] = 0.0

    num_ref[0, 0] += num_s
    den_ref[0, 0] += den_s


def _final_reduce(num_ref, den_ref, ntc_ref, dtc_ref, out_ref):
    s = ((jnp.sum(num_ref[...]) + ntc_ref[0, 0])
         / (jnp.sum(den_ref[...]) + dtc_ref[0, 0]))
    out_ref[...] = jnp.full((1, 1), s, jnp.float32)


def kernel(y_hat, efs_time, efs):
    y_hat = jnp.squeeze(y_hat).astype(jnp.float32)
    efs_time = efs_time.astype(jnp.float32)
    efs_f = efs.astype(jnp.float32)
    num, den = _pairwise_sc(y_hat, efs_time, efs_f)
    p2 = jnp.concatenate([y_hat, y_hat]).reshape(1, 2 * B)
    t2 = jnp.concatenate([efs_time, efs_time]).reshape(1, 2 * B)
    e2 = jnp.concatenate([efs_f, efs_f]).reshape(1, 2 * B)
    pcol = y_hat.reshape(B, 1)
    tcol = efs_time.reshape(B, 1)
    ecol = efs_f.reshape(B, 1)
    piece_specs = [
        pl.BlockSpec((1, TC_BLK), lambda k, c=c: (0, SPLIT_ROWS // TC_BLK + k + c))
        for c in range(NPIECE)
    ]
    col_spec = pl.BlockSpec((TC_BLK, 1), lambda k: (SPLIT_ROWS // TC_BLK + k, 0))
    ntc, dtc = pl.pallas_call(
        _tc_band,
        grid=((B - SPLIT_ROWS) // TC_BLK,),
        in_specs=(piece_specs * 3) + [col_spec, col_spec, col_spec],
        out_specs=[
            pl.BlockSpec(memory_space=pltpu.SMEM),
            pl.BlockSpec(memory_space=pltpu.SMEM),
        ],
        out_shape=[
            jax.ShapeDtypeStruct((1, 1), jnp.float32),
            jax.ShapeDtypeStruct((1, 1), jnp.float32),
        ],
    )(*([p2] * NPIECE), *([t2] * NPIECE), *([e2] * NPIECE), pcol, tcol, ecol)
    out = pl.pallas_call(
        _final_reduce,
        in_specs=[
            pl.BlockSpec(memory_space=pltpu.VMEM),
            pl.BlockSpec(memory_space=pltpu.VMEM),
            pl.BlockSpec(memory_space=pltpu.SMEM),
            pl.BlockSpec(memory_space=pltpu.SMEM),
        ],
        out_shape=jax.ShapeDtypeStruct((1, 1), jnp.float32),
    )(num, den, ntc, dtc)
    return out[0, 0]


# speedup vs baseline: 1.1778x; 1.0002x over previous
import functools

import jax
import jax.numpy as jnp
from jax import lax
from jax.experimental import pallas as pl
from jax.experimental.pallas import tpu as pltpu
from jax.experimental.pallas import tpu_sc as plsc

B = 4096
MARGIN = 0.5
L = 16
NC = 2
NS = 16
NW = NC * NS
SPLIT_ROWS = 1536
ROWS_PER_W = SPLIT_ROWS // NW
NCHUNK = B // L
TC_BLK = 512
INF = float("inf")

_mesh = plsc.VectorSubcoreMesh(core_axis_name="c", subcore_axis_name="s")


@functools.partial(
    pl.kernel,
    mesh=_mesh,
    out_type=[
        jax.ShapeDtypeStruct((NW, L), jnp.float32),
        jax.ShapeDtypeStruct((NW, L), jnp.float32),
    ],
    scratch_types=[
        pltpu.VMEM((B,), jnp.float32),
        pltpu.VMEM((B,), jnp.float32),
        pltpu.VMEM((B + L,), jnp.float32),
        pltpu.VMEM((B,), jnp.float32),
        pltpu.VMEM((B,), jnp.float32),
        pltpu.VMEM((L,), jnp.float32),
        pltpu.VMEM((L,), jnp.float32),
    ],
)
def _pairwise_sc(p_hbm, t_hbm, e_hbm, num_hbm, den_hbm,
                 pv, tv, ev, t0v, t1v, nv, dv):
    cid = lax.axis_index("c")
    sid = lax.axis_index("s")
    wid = sid * NC + cid

    pltpu.sync_copy(p_hbm, pv)
    pltpu.sync_copy(t_hbm, tv)
    pltpu.sync_copy(e_hbm, ev.at[pl.ds(0, B)])

    lanes = lax.iota(jnp.int32, L)
    zeros = jnp.zeros((L,), jnp.float32)
    ones = jnp.ones((L,), jnp.float32)
    infs = jnp.full((L,), INF, jnp.float32)

    @plsc.parallel_loop(0, NCHUNK, unroll=8)
    def prep_body(c):
        b2 = c * L
        t_c = tv[pl.ds(b2, L)]
        e_c = ev[pl.ds(b2, L)] > 0.0
        t0v[pl.ds(b2, L)] = jnp.where(e_c, t_c, infs)
        t1v[pl.ds(b2, L)] = jnp.where(e_c, infs, t_c)

    def contrib(p_i, t_i, e_i, p_j, t_j, e_j):
        lt = t_i < t_j
        gt = t_j < t_i
        d = p_i - p_j
        yd = jnp.where(lt, d, -d)
        h = jnp.maximum(MARGIN - yd, 0.0)
        m = jnp.where(lt, e_i, jnp.where(gt, e_j, e_i * e_j))
        return h * m, m

    nv[...] = zeros
    dv[...] = zeros

    def row_body(k, dummy):
        i = wid + NW * k
        cc = i // L
        off = i - cc * L
        base = cc * L
        p_c = pv[pl.ds(base, L)]
        t_c = tv[pl.ds(base, L)]
        e_c = ev[pl.ds(base, L)]
        lane = jnp.full((L,), off, dtype=jnp.int32)
        p_i = p_c.at[lane].get(mode="promise_in_bounds")
        t_i = t_c.at[lane].get(mode="promise_in_bounds")
        e_i = e_c.at[lane].get(mode="promise_in_bounds")
        mi = MARGIN - p_i
        ma = MARGIN + p_i
        hm, m = contrib(p_i, t_i, e_i, p_c, t_c, e_c)
        tri = lanes > off

        cx = ((cc + NCHUNK // 2) & (NCHUNK - 1)) * L
        hm_x, m_x = contrib(p_i, t_i, e_i,
                            pv[pl.ds(cx, L)], tv[pl.ds(cx, L)], ev[pl.ds(cx, L)])
        lim = off - jnp.where(i >= B // 2, 1, 0)
        keep_x = lanes <= lim

        ce = ((cc + NCHUNK // 2 - 1) & (NCHUNK - 1)) * L
        hm_e, m_e = contrib(p_i, t_i, e_i,
                            pv[pl.ds(ce, L)], tv[pl.ds(ce, L)], ev[pl.ds(ce, L)])

        nv[...] = (nv[...] + jnp.where(tri, hm, zeros)
                   + jnp.where(keep_x, hm_x, zeros) + hm_e)
        dv[...] = (dv[...] + jnp.where(tri, m, zeros)
                   + jnp.where(keep_x, m_x, zeros) + m_e)

        HALF = NCHUNK // 4 - 1

        def rows_with_event(_):
            @plsc.parallel_loop(1, HALF + 1, unroll=8,
                                carry=(zeros, zeros, zeros, zeros))
            def loop1(c, carry2):
                na, da, nb, db = carry2
                bA = ((cc + c) & (NCHUNK - 1)) * L
                bB = ((cc + c + HALF) & (NCHUNK - 1)) * L
                pA = pv[pl.ds(bA, L)]
                tA = tv[pl.ds(bA, L)]
                t1A = t1v[pl.ds(bA, L)]
                pB = pv[pl.ds(bB, L)]
                tB = tv[pl.ds(bB, L)]
                t1B = t1v[pl.ds(bB, L)]
                ltA = t_i < tA
                mA = t_i < t1A
                hA = jnp.maximum(jnp.where(ltA, mi + pA, ma - pA), 0.0)
                ltB = t_i < tB
                mB = t_i < t1B
                hB = jnp.maximum(jnp.where(ltB, mi + pB, ma - pB), 0.0)
                return (na + jnp.where(mA, hA, zeros),
                        da + jnp.where(mA, ones, zeros),
                        nb + jnp.where(mB, hB, zeros),
                        db + jnp.where(mB, ones, zeros))

            na, da, nb, db = loop1
            nv[...] = nv[...] + na + nb
            dv[...] = dv[...] + da + db

        def rows_without_event(_):
            @plsc.parallel_loop(1, HALF + 1, unroll=8,
                                carry=(zeros, zeros, zeros, zeros))
            def loop0(c, carry2):
                na, da, nb, db = carry2
                bA = ((cc + c) & (NCHUNK - 1)) * L
                bB = ((cc + c + HALF) & (NCHUNK - 1)) * L
                pA = pv[pl.ds(bA, L)]
                t0A = t0v[pl.ds(bA, L)]
                pB = pv[pl.ds(bB, L)]
                t0B = t0v[pl.ds(bB, L)]
                mA = t0A < t_i
                hA = jnp.maximum(ma - pA, 0.0)
                mB = t0B < t_i
                hB = jnp.maximum(ma - pB, 0.0)
                return (na + jnp.where(mA, hA, zeros),
                        da + jnp.where(mA, ones, zeros),
                        nb + jnp.where(mB, hB, zeros),
                        db + jnp.where(mB, ones, zeros))

            na, da, nb, db = loop0
            nv[...] = nv[...] + na + nb
            dv[...] = dv[...] + da + db

        has_event = ev[pl.ds(i, L)][0] > 0.0
        lax.cond(has_event, rows_with_event, rows_without_event, 0)
        return dummy

    lax.fori_loop(0, ROWS_PER_W, row_body, 0)
    pltpu.sync_copy(nv, num_hbm.at[wid])
    pltpu.sync_copy(dv, den_hbm.at[wid])


NPIECE = B // 2 // TC_BLK + 1


def _tc_band(*refs):
    ps = refs[0:NPIECE]
    ts = refs[NPIECE:2 * NPIECE]
    es = refs[2 * NPIECE:3 * NPIECE]
    pcol, tcol, ecol, num_ref, den_ref = refs[3 * NPIECE:]
    k = pl.program_id(0)
    base = SPLIT_ROWS + k * TC_BLK
    p_i = pcol[...]
    t_i = tcol[...]
    e_i = ecol[...]
    mi = MARGIN - p_i
    ma = MARGIN + p_i
    ei_pos = e_i > 0.0
    r = lax.broadcasted_iota(jnp.int32, (TC_BLK, 1), 0)
    lim = r + B // 2 - jnp.where(base + r >= B // 2, 1, 0)
    num_s = jnp.zeros((), jnp.float32)
    den_s = jnp.zeros((), jnp.float32)
    for c in range(NPIECE):
        pj = ps[c][...]
        tj = ts[c][...]
        ej = es[c][...]
        ebj = ej > 0.0
        t0s = jnp.where(ebj, tj, INF)
        t1s = jnp.where(ebj, INF, tj)
        s = c * TC_BLK + lax.broadcasted_iota(jnp.int32, (1, TC_BLK), 1)
        keep = (s > r) & (s <= lim)
        lt = t_i < tj
        m = ((ei_pos & (t_i < t1s)) | (~ei_pos & (t0s < t_i))) & keep
        h = jnp.maximum(jnp.where(lt, mi + pj, ma - pj), 0.0)
        num_s += jnp.sum(jnp.where(m, h, 0.0))
        den_s += jnp.sum(jnp.where(m, 1.0, 0.0))

    @pl.when(k == 0)
    def _():
        num_ref[0, 0] = 0.0
        den_ref[0, 0] = 0.0

    num_ref[0, 0] += num_s
    den_ref[0, 0] += den_s


def _final_reduce(num_ref, den_ref, ntc_ref, dtc_ref, out_ref):
    s = ((jnp.sum(num_ref[...]) + ntc_ref[0, 0])
         / (jnp.sum(den_ref[...]) + dtc_ref[0, 0]))
    out_ref[...] = jnp.full((1, 1), s, jnp.float32)


def kernel(y_hat, efs_time, efs):
    y_hat = jnp.squeeze(y_hat).astype(jnp.float32)
    efs_time = efs_time.astype(jnp.float32)
    efs_f = efs.astype(jnp.float32)
    num, den = _pairwise_sc(y_hat, efs_time, efs_f)
    p2 = jnp.concatenate([y_hat, y_hat]).reshape(1, 2 * B)
    t2 = jnp.concatenate([efs_time, efs_time]).reshape(1, 2 * B)
    e2 = jnp.concatenate([efs_f, efs_f]).reshape(1, 2 * B)
    pcol = y_hat.reshape(B, 1)
    tcol = efs_time.reshape(B, 1)
    ecol = efs_f.reshape(B, 1)
    piece_specs = [
        pl.BlockSpec((1, TC_BLK), lambda k, c=c: (0, SPLIT_ROWS // TC_BLK + k + c))
        for c in range(NPIECE)
    ]
    col_spec = pl.BlockSpec((TC_BLK, 1), lambda k: (SPLIT_ROWS // TC_BLK + k, 0))
    ntc, dtc = pl.pallas_call(
        _tc_band,
        grid=((B - SPLIT_ROWS) // TC_BLK,),
        in_specs=(piece_specs * 3) + [col_spec, col_spec, col_spec],
        out_specs=[
            pl.BlockSpec(memory_space=pltpu.SMEM),
            pl.BlockSpec(memory_space=pltpu.SMEM),
        ],
        out_shape=[
            jax.ShapeDtypeStruct((1, 1), jnp.float32),
            jax.ShapeDtypeStruct((1, 1), jnp.float32),
        ],
    )(*([p2] * NPIECE), *([t2] * NPIECE), *([e2] * NPIECE), pcol, tcol, ecol)
    out = pl.pallas_call(
        _final_reduce,
        in_specs=[
            pl.BlockSpec(memory_space=pltpu.VMEM),
            pl.BlockSpec(memory_space=pltpu.VMEM),
            pl.BlockSpec(memory_space=pltpu.SMEM),
            pl.BlockSpec(memory_space=pltpu.SMEM),
        ],
        out_shape=jax.ShapeDtypeStruct((1, 1), jnp.float32),
    )(num, den, ntc, dtc)
    return out[0, 0]


# speedup vs baseline: 1.2818x; 1.0883x over previous
import functools

import jax
import jax.numpy as jnp
from jax import lax
from jax.experimental import pallas as pl
from jax.experimental.pallas import tpu as pltpu
from jax.experimental.pallas import tpu_sc as plsc

B = 4096
MARGIN = 0.5
L = 16
NC = 2
NS = 16
NW = NC * NS
SPLIT_ROWS = 2048
ROWS_PER_W = SPLIT_ROWS // NW
NCHUNK = B // L
TC_BLK = 512
INF = float("inf")

_mesh = plsc.VectorSubcoreMesh(core_axis_name="c", subcore_axis_name="s")


@functools.partial(
    pl.kernel,
    mesh=_mesh,
    out_type=[
        jax.ShapeDtypeStruct((NW, L), jnp.float32),
        jax.ShapeDtypeStruct((NW, L), jnp.float32),
    ],
    scratch_types=[
        pltpu.VMEM((B,), jnp.float32),
        pltpu.VMEM((B,), jnp.float32),
        pltpu.VMEM((B + L,), jnp.float32),
        pltpu.VMEM((B,), jnp.float32),
        pltpu.VMEM((B,), jnp.float32),
        pltpu.VMEM((L,), jnp.float32),
        pltpu.VMEM((L,), jnp.float32),
    ],
)
def _pairwise_sc(p_hbm, t_hbm, e_hbm, num_hbm, den_hbm,
                 pv, tv, ev, t0v, t1v, nv, dv):
    cid = lax.axis_index("c")
    sid = lax.axis_index("s")
    wid = sid * NC + cid

    pltpu.sync_copy(p_hbm, pv)
    pltpu.sync_copy(t_hbm, tv)
    pltpu.sync_copy(e_hbm, ev.at[pl.ds(0, B)])

    lanes = lax.iota(jnp.int32, L)
    zeros = jnp.zeros((L,), jnp.float32)
    ones = jnp.ones((L,), jnp.float32)
    infs = jnp.full((L,), INF, jnp.float32)

    @plsc.parallel_loop(0, NCHUNK, unroll=8)
    def prep_body(c):
        b2 = c * L
        t_c = tv[pl.ds(b2, L)]
        e_c = ev[pl.ds(b2, L)] > 0.0
        t0v[pl.ds(b2, L)] = jnp.where(e_c, t_c, infs)
        t1v[pl.ds(b2, L)] = jnp.where(e_c, infs, t_c)

    def contrib(p_i, t_i, e_i, p_j, t_j, e_j):
        lt = t_i < t_j
        gt = t_j < t_i
        d = p_i - p_j
        yd = jnp.where(lt, d, -d)
        h = jnp.maximum(MARGIN - yd, 0.0)
        m = jnp.where(lt, e_i, jnp.where(gt, e_j, e_i * e_j))
        return h * m, m

    nv[...] = zeros
    dv[...] = zeros

    def row_body(k, dummy):
        i = wid + NW * k
        cc = i // L
        off = i - cc * L
        base = cc * L
        p_c = pv[pl.ds(base, L)]
        t_c = tv[pl.ds(base, L)]
        e_c = ev[pl.ds(base, L)]
        lane = jnp.full((L,), off, dtype=jnp.int32)
        p_i = p_c.at[lane].get(mode="promise_in_bounds")
        t_i = t_c.at[lane].get(mode="promise_in_bounds")
        e_i = e_c.at[lane].get(mode="promise_in_bounds")
        mi = MARGIN - p_i
        ma = MARGIN + p_i
        hm, m = contrib(p_i, t_i, e_i, p_c, t_c, e_c)
        tri = lanes > off

        cx = ((cc + NCHUNK // 2) & (NCHUNK - 1)) * L
        hm_x, m_x = contrib(p_i, t_i, e_i,
                            pv[pl.ds(cx, L)], tv[pl.ds(cx, L)], ev[pl.ds(cx, L)])
        lim = off - jnp.where(i >= B // 2, 1, 0)
        keep_x = lanes <= lim

        ce = ((cc + NCHUNK // 2 - 1) & (NCHUNK - 1)) * L
        hm_e, m_e = contrib(p_i, t_i, e_i,
                            pv[pl.ds(ce, L)], tv[pl.ds(ce, L)], ev[pl.ds(ce, L)])

        nv[...] = (nv[...] + jnp.where(tri, hm, zeros)
                   + jnp.where(keep_x, hm_x, zeros) + hm_e)
        dv[...] = (dv[...] + jnp.where(tri, m, zeros)
                   + jnp.where(keep_x, m_x, zeros) + m_e)

        TRI = 42

        def rows_with_event(_):
            @plsc.parallel_loop(1, TRI + 1, unroll=6,
                                carry=(zeros, zeros, zeros, zeros, zeros, zeros))
            def loop1(c, carry2):
                na, da, nb, db, ng, dg = carry2
                bA = ((cc + c) & (NCHUNK - 1)) * L
                bB = ((cc + c + TRI) & (NCHUNK - 1)) * L
                bC = ((cc + c + 2 * TRI) & (NCHUNK - 1)) * L
                pA = pv[pl.ds(bA, L)]
                tA = tv[pl.ds(bA, L)]
                t1A = t1v[pl.ds(bA, L)]
                pB = pv[pl.ds(bB, L)]
                tB = tv[pl.ds(bB, L)]
                t1B = t1v[pl.ds(bB, L)]
                pC = pv[pl.ds(bC, L)]
                tC = tv[pl.ds(bC, L)]
                t1C = t1v[pl.ds(bC, L)]
                ltA = t_i < tA
                mA = t_i < t1A
                hA = jnp.maximum(jnp.where(ltA, mi + pA, ma - pA), 0.0)
                ltB = t_i < tB
                mB = t_i < t1B
                hB = jnp.maximum(jnp.where(ltB, mi + pB, ma - pB), 0.0)
                ltC = t_i < tC
                mC = t_i < t1C
                hC = jnp.maximum(jnp.where(ltC, mi + pC, ma - pC), 0.0)
                return (na + jnp.where(mA, hA, zeros),
                        da + jnp.where(mA, ones, zeros),
                        nb + jnp.where(mB, hB, zeros),
                        db + jnp.where(mB, ones, zeros),
                        ng + jnp.where(mC, hC, zeros),
                        dg + jnp.where(mC, ones, zeros))

            na, da, nb, db, ng, dg = loop1
            nv[...] = nv[...] + na + nb + ng
            dv[...] = dv[...] + da + db + dg

        def rows_without_event(_):
            @plsc.parallel_loop(1, TRI + 1, unroll=6,
                                carry=(zeros, zeros, zeros, zeros, zeros, zeros))
            def loop0(c, carry2):
                na, da, nb, db, ng, dg = carry2
                bA = ((cc + c) & (NCHUNK - 1)) * L
                bB = ((cc + c + TRI) & (NCHUNK - 1)) * L
                bC = ((cc + c + 2 * TRI) & (NCHUNK - 1)) * L
                pA = pv[pl.ds(bA, L)]
                t0A = t0v[pl.ds(bA, L)]
                pB = pv[pl.ds(bB, L)]
                t0B = t0v[pl.ds(bB, L)]
                pC = pv[pl.ds(bC, L)]
                t0C = t0v[pl.ds(bC, L)]
                mA = t0A < t_i
                hA = jnp.maximum(ma - pA, 0.0)
                mB = t0B < t_i
                hB = jnp.maximum(ma - pB, 0.0)
                mC = t0C < t_i
                hC = jnp.maximum(ma - pC, 0.0)
                return (na + jnp.where(mA, hA, zeros),
                        da + jnp.where(mA, ones, zeros),
                        nb + jnp.where(mB, hB, zeros),
                        db + jnp.where(mB, ones, zeros),
                        ng + jnp.where(mC, hC, zeros),
                        dg + jnp.where(mC, ones, zeros))

            na, da, nb, db, ng, dg = loop0
            nv[...] = nv[...] + na + nb + ng
            dv[...] = dv[...] + da + db + dg

        has_event = ev[pl.ds(i, L)][0] > 0.0
        lax.cond(has_event, rows_with_event, rows_without_event, 0)
        return dummy

    lax.fori_loop(0, ROWS_PER_W, row_body, 0)
    pltpu.sync_copy(nv, num_hbm.at[wid])
    pltpu.sync_copy(dv, den_hbm.at[wid])


NPIECE = B // 2 // TC_BLK + 1


def _tc_band(*refs):
    ps = refs[0:NPIECE]
    ts = refs[NPIECE:2 * NPIECE]
    es = refs[2 * NPIECE:3 * NPIECE]
    pcol, tcol, ecol, num_ref, den_ref = refs[3 * NPIECE:]
    k = pl.program_id(0)
    base = SPLIT_ROWS + k * TC_BLK
    p_i = pcol[...]
    t_i = tcol[...]
    e_i = ecol[...]
    mi = MARGIN - p_i
    ma = MARGIN + p_i
    ei_pos = e_i > 0.0
    r = lax.broadcasted_iota(jnp.int32, (TC_BLK, 1), 0)
    lim = r + B // 2 - jnp.where(base + r >= B // 2, 1, 0)
    num_s = jnp.zeros((), jnp.float32)
    den_s = jnp.zeros((), jnp.float32)
    for c in range(NPIECE):
        pj = ps[c][...]
        tj = ts[c][...]
        ej = es[c][...]
        ebj = ej > 0.0
        t0s = jnp.where(ebj, tj, INF)
        t1s = jnp.where(ebj, INF, tj)
        s = c * TC_BLK + lax.broadcasted_iota(jnp.int32, (1, TC_BLK), 1)
        keep = (s > r) & (s <= lim)
        lt = t_i < tj
        m = ((ei_pos & (t_i < t1s)) | (~ei_pos & (t0s < t_i))) & keep
        h = jnp.maximum(jnp.where(lt, mi + pj, ma - pj), 0.0)
        num_s += jnp.sum(jnp.where(m, h, 0.0))
        den_s += jnp.sum(jnp.where(m, 1.0, 0.0))

    @pl.when(k == 0)
    def _():
        num_ref[0, 0] = 0.0
        den_ref[0, 0] = 0.0

    num_ref[0, 0] += num_s
    den_ref[0, 0] += den_s


def _final_reduce(num_ref, den_ref, ntc_ref, dtc_ref, out_ref):
    s = ((jnp.sum(num_ref[...]) + ntc_ref[0, 0])
         / (jnp.sum(den_ref[...]) + dtc_ref[0, 0]))
    out_ref[...] = jnp.full((1, 1), s, jnp.float32)


def kernel(y_hat, efs_time, efs):
    y_hat = jnp.squeeze(y_hat).astype(jnp.float32)
    efs_time = efs_time.astype(jnp.float32)
    efs_f = efs.astype(jnp.float32)
    num, den = _pairwise_sc(y_hat, efs_time, efs_f)
    p2 = jnp.concatenate([y_hat, y_hat]).reshape(1, 2 * B)
    t2 = jnp.concatenate([efs_time, efs_time]).reshape(1, 2 * B)
    e2 = jnp.concatenate([efs_f, efs_f]).reshape(1, 2 * B)
    pcol = y_hat.reshape(B, 1)
    tcol = efs_time.reshape(B, 1)
    ecol = efs_f.reshape(B, 1)
    piece_specs = [
        pl.BlockSpec((1, TC_BLK), lambda k, c=c: (0, SPLIT_ROWS // TC_BLK + k + c))
        for c in range(NPIECE)
    ]
    col_spec = pl.BlockSpec((TC_BLK, 1), lambda k: (SPLIT_ROWS // TC_BLK + k, 0))
    ntc, dtc = pl.pallas_call(
        _tc_band,
        grid=((B - SPLIT_ROWS) // TC_BLK,),
        in_specs=(piece_specs * 3) + [col_spec, col_spec, col_spec],
        out_specs=[
            pl.BlockSpec(memory_space=pltpu.SMEM),
            pl.BlockSpec(memory_space=pltpu.SMEM),
        ],
        out_shape=[
            jax.ShapeDtypeStruct((1, 1), jnp.float32),
            jax.ShapeDtypeStruct((1, 1), jnp.float32),
        ],
    )(*([p2] * NPIECE), *([t2] * NPIECE), *([e2] * NPIECE), pcol, tcol, ecol)
    out = pl.pallas_call(
        _final_reduce,
        in_specs=[
            pl.BlockSpec(memory_space=pltpu.VMEM),
            pl.BlockSpec(memory_space=pltpu.VMEM),
            pl.BlockSpec(memory_space=pltpu.SMEM),
            pl.BlockSpec(memory_space=pltpu.SMEM),
        ],
        out_shape=jax.ShapeDtypeStruct((1, 1), jnp.float32),
    )(num, den, ntc, dtc)
    return out[0, 0]
